# Initial kernel scaffold; baseline (speedup 1.0000x reference)
import jax
import jax.numpy as jnp
from jax.experimental import pallas as pl


def kernel(x, edge_index, W1, b1, W2, b2):
    raise NotImplementedError("write your pallas kernel here")


# speedup vs baseline: 12.0472x; 12.0472x over previous
import functools

import jax
import jax.numpy as jnp
from jax import lax
from jax.experimental import pallas as pl
from jax.experimental.pallas import tpu as pltpu
from jax.experimental.pallas import tpu_sc as plsc

N_NODES = 10000
N_EDGES = 320000
D = 128

NC = 2
NS = 16
NW = NC * NS
CHUNK = 128
CPT = -(-N_EDGES // (NW * CHUNK))
SLAB_CPT = CPT + 1
NP = 10240
STRIPE = NP // NS
DEG_W = 16

_sc_mesh = plsc.VectorSubcoreMesh(core_axis_name="c", subcore_axis_name="s")


@functools.partial(
    pl.kernel,
    out_type=jax.ShapeDtypeStruct((NC, NP, DEG_W), jnp.float32),
    mesh=_sc_mesh,
    scratch_types=[
        pltpu.VMEM((CHUNK,), jnp.int32),
        pltpu.VMEM((CHUNK, DEG_W), jnp.float32),
        pltpu.VMEM_SHARED((NP, DEG_W), jnp.float32),
    ],
)
def _deg_kernel(slab_hbm, ones_hbm, zeros_hbm, out_hbm, dst_i, ones_v, acc_sh):
    c = lax.axis_index("c")
    s = lax.axis_index("s")

    pltpu.sync_copy(ones_hbm, ones_v)
    pltpu.sync_copy(zeros_hbm.at[pl.ds(s * STRIPE, STRIPE), :],
                    acc_sh.at[pl.ds(s * STRIPE, STRIPE), :])
    plsc.subcore_barrier()

    def body(j, _):
        pltpu.sync_copy(slab_hbm.at[c, s, j, 1, 0], dst_i)
        pltpu.sync_copy(ones_v, acc_sh.at[dst_i], add=True)
        return 0

    lax.fori_loop(0, CPT, body, 0)
    plsc.subcore_barrier()

    pltpu.sync_copy(acc_sh.at[pl.ds(s * STRIPE, STRIPE), :],
                    out_hbm.at[c, pl.ds(s * STRIPE, STRIPE), :])


@functools.partial(
    pl.kernel,
    out_type=jax.ShapeDtypeStruct((NC, NP, D), jnp.float32),
    mesh=_sc_mesh,
    scratch_types=[
        pltpu.VMEM((CHUNK,), jnp.int32),
        pltpu.VMEM((CHUNK,), jnp.int32),
        pltpu.VMEM((CHUNK,), jnp.int32),
        pltpu.VMEM((CHUNK,), jnp.int32),
        pltpu.VMEM((CHUNK, D), jnp.float32),
        pltpu.VMEM((CHUNK, D), jnp.float32),
        pltpu.VMEM_SHARED((NP, D), jnp.float32),
        pltpu.SemaphoreType.DMA,
        pltpu.SemaphoreType.DMA,
        pltpu.SemaphoreType.DMA,
        pltpu.SemaphoreType.DMA,
        pltpu.SemaphoreType.DMA,
        pltpu.SemaphoreType.DMA,
    ],
)
def _scatter_kernel(slab_hbm, y_hbm, zeros_hbm, out_hbm,
                    sa, da, sb, db, buf_a, buf_b, acc_sh,
                    sem_a, sem_b, sem_ia, sem_ib, sem_sa, sem_sb):
    c = lax.axis_index("c")
    s = lax.axis_index("s")

    pltpu.sync_copy(zeros_hbm.at[pl.ds(s * STRIPE, STRIPE), :],
                    acc_sh.at[pl.ds(s * STRIPE, STRIPE), :])
    plsc.subcore_barrier()

    def load_idx(j, si, di, sem):
        pltpu.async_copy(slab_hbm.at[c, s, j, 0, 0], si, sem)
        pltpu.async_copy(slab_hbm.at[c, s, j, 1, 0], di, sem)

    def wait_idx(j, si, di, sem):
        pltpu.make_async_copy(slab_hbm.at[c, s, j, 0, 0], si, sem).wait()
        pltpu.make_async_copy(slab_hbm.at[c, s, j, 1, 0], di, sem).wait()

    load_idx(0, sa, da, sem_ia)
    wait_idx(0, sa, da, sem_ia)
    pltpu.async_copy(y_hbm.at[sa], buf_a, sem_a)
    load_idx(1, sb, db, sem_ib)

    def body(t, _):
        ja = 2 * t
        wait_idx(ja + 1, sb, db, sem_ib)
        pltpu.async_copy(y_hbm.at[sb], buf_b, sem_b)
        pltpu.make_async_copy(y_hbm.at[sa], buf_a, sem_a).wait()
        pltpu.async_copy(buf_a, acc_sh.at[da], sem_sa, add=True)
        pltpu.make_async_copy(y_hbm.at[sb], buf_b, sem_b).wait()
        pltpu.make_async_copy(buf_a, acc_sh.at[da], sem_sa).wait()
        pltpu.async_copy(buf_b, acc_sh.at[db], sem_sb, add=True)
        load_idx(ja + 2, sa, da, sem_ia)
        wait_idx(ja + 2, sa, da, sem_ia)
        pltpu.async_copy(y_hbm.at[sa], buf_a, sem_a)
        pltpu.make_async_copy(buf_b, acc_sh.at[db], sem_sb).wait()
        load_idx(ja + 3, sb, db, sem_ib)
        return 0

    lax.fori_loop(0, (CPT - 1) // 2, body, 0)
    wait_idx(CPT, sb, db, sem_ib)
    pltpu.make_async_copy(y_hbm.at[sa], buf_a, sem_a).wait()
    pltpu.sync_copy(buf_a, acc_sh.at[da], add=True)

    plsc.subcore_barrier()
    pltpu.sync_copy(acc_sh.at[pl.ds(s * STRIPE, STRIPE), :],
                    out_hbm.at[c, pl.ds(s * STRIPE, STRIPE), :])


@functools.partial(
    pl.kernel,
    out_type=jax.ShapeDtypeStruct((NW * CPT * CHUNK, D), jnp.float32),
    mesh=_sc_mesh,
    scratch_types=[
        pltpu.VMEM((CHUNK,), jnp.int32),
        pltpu.VMEM((CHUNK, D), jnp.float32),
        pltpu.SemaphoreType.DMA,
    ],
)
def _gather_dbg_kernel(slab_hbm, y_hbm, out_hbm, sa, buf_a, sem_a):
    c = lax.axis_index("c")
    s = lax.axis_index("s")
    w = c * NS + s

    def body(j, _):
        pltpu.sync_copy(slab_hbm.at[c, s, j, 0, 0], sa)
        pltpu.async_copy(y_hbm.at[sa], buf_a, sem_a).wait()
        base = (w * CPT + j) * CHUNK
        pltpu.sync_copy(buf_a, out_hbm.at[pl.ds(base, CHUNK), :])
        return 0

    lax.fori_loop(0, CPT, body, 0)


_RB = 1024
_GRID = NP // _RB


def _dis_y_body(deg_ref, x_ref, w_ref, dis_ref, y_ref):
    d = deg_ref[0, :, 0:1] + deg_ref[1, :, 0:1] + 1.0
    dis = lax.rsqrt(d)
    dis_ref[...] = dis
    y_ref[...] = jnp.dot(x_ref[...], w_ref[...],
                         preferred_element_type=jnp.float32) * dis


def _tc_dis_y(deg, x, w):
    return pl.pallas_call(
        _dis_y_body,
        grid=(_GRID,),
        in_specs=[
            pl.BlockSpec((NC, _RB, D), lambda i: (0, i, 0)),
            pl.BlockSpec((_RB, D), lambda i: (i, 0)),
            pl.BlockSpec((D, D), lambda i: (0, 0)),
        ],
        out_specs=[
            pl.BlockSpec((_RB, 1), lambda i: (i, 0)),
            pl.BlockSpec((_RB, D), lambda i: (i, 0)),
        ],
        out_shape=[
            jax.ShapeDtypeStruct((NP, 1), jnp.float32),
            jax.ShapeDtypeStruct((NP, D), jnp.float32),
        ],
    )(deg, x, w)


def _mid_body(z_ref, y_ref, dis_ref, b_ref, w_ref, out_ref):
    zb = z_ref[0] + z_ref[1] + y_ref[...]
    h = jnp.maximum(dis_ref[...] * zb + b_ref[...], 0.0)
    out_ref[...] = jnp.dot(h, w_ref[...],
                           preferred_element_type=jnp.float32) * dis_ref[...]


def _tc_mid(z, y, dis, b, w):
    return pl.pallas_call(
        _mid_body,
        grid=(_GRID,),
        in_specs=[
            pl.BlockSpec((NC, _RB, D), lambda i: (0, i, 0)),
            pl.BlockSpec((_RB, D), lambda i: (i, 0)),
            pl.BlockSpec((_RB, 1), lambda i: (i, 0)),
            pl.BlockSpec((1, D), lambda i: (0, 0)),
            pl.BlockSpec((D, D), lambda i: (0, 0)),
        ],
        out_specs=pl.BlockSpec((_RB, D), lambda i: (i, 0)),
        out_shape=jax.ShapeDtypeStruct((NP, D), jnp.float32),
    )(z, y, dis, b, w)


def _final_body(z_ref, y_ref, dis_ref, b_ref, out_ref):
    out_ref[...] = dis_ref[...] * (z_ref[0] + z_ref[1] + y_ref[...]) + b_ref[...]


def _tc_final(z, y, dis, b):
    return pl.pallas_call(
        _final_body,
        grid=(_GRID,),
        in_specs=[
            pl.BlockSpec((NC, _RB, D), lambda i: (0, i, 0)),
            pl.BlockSpec((_RB, D), lambda i: (i, 0)),
            pl.BlockSpec((_RB, 1), lambda i: (i, 0)),
            pl.BlockSpec((1, D), lambda i: (0, 0)),
        ],
        out_specs=pl.BlockSpec((_RB, D), lambda i: (i, 0)),
        out_shape=jax.ShapeDtypeStruct((NP, D), jnp.float32),
    )(z, y, dis, b)


def kernel(x, edge_index, W1, b1, W2, b2):
    ei = edge_index.astype(jnp.int32)
    e_proc = NW * CPT * CHUNK
    pad = e_proc - N_EDGES
    src = jnp.concatenate([ei[0], jnp.zeros((pad,), jnp.int32)])
    dst = jnp.concatenate([ei[1], jnp.full((pad,), N_NODES, jnp.int32)])
    slab = jnp.stack(
        [src.reshape(NW, CPT, CHUNK), dst.reshape(NW, CPT, CHUNK)], axis=2)
    pad_chunk = jnp.broadcast_to(
        jnp.stack([jnp.zeros((CHUNK,), jnp.int32),
                   jnp.full((CHUNK,), N_NODES, jnp.int32)]),
        (NW, 1, 2, CHUNK))
    slab = jnp.concatenate([slab, pad_chunk], axis=1)
    slab = slab.reshape(NC, NS, SLAB_CPT, 2, 1, CHUNK)

    xp = jnp.pad(x, ((0, NP - N_NODES), (0, 0)))
    ones_deg = jnp.ones((CHUNK, DEG_W), jnp.float32)
    zeros_deg = jnp.zeros((NP, DEG_W), jnp.float32)
    zeros_z = jnp.zeros((NP, D), jnp.float32)
    b1r = b1.reshape(1, D)
    b2r = b2.reshape(1, D)

    ones_y = jnp.ones((NP, D), jnp.float32)
    degp = _scatter_kernel(slab, ones_y, zeros_z)
    dis, y1 = _tc_dis_y(degp, xp, W1)
    z1 = _scatter_kernel(slab, y1, zeros_z)
    y2 = _tc_mid(z1, y1, dis, b1r, W2)
    z2 = _scatter_kernel(slab, y2, zeros_z)
    out = _tc_final(z2, y2, dis, b2r)
    return out[:N_NODES]
